# Initial kernel scaffold; baseline (speedup 1.0000x reference)
import jax
import jax.numpy as jnp
from jax.experimental import pallas as pl


def kernel(x, edge_index, pooled_x, pooled_edge_index, unpool_info, W0, b0, W1, b1, W2, b2):
    raise NotImplementedError("write your pallas kernel here")


# speedup vs baseline: 4.3129x; 4.3129x over previous
import jax
import jax.numpy as jnp
from jax import lax
from jax.experimental import pallas as pl
from jax.experimental.pallas import tpu as pltpu
from jax.experimental.pallas import tpu_sc as plsc

NC = 2
NS = 16
NW = NC * NS
L = 16
C = 128

D = 128
N_FULL = 10000
N_POOL = 5000
NPAD_FULL = 10240
NPAD_POOL = 5120
UP_PAD = NPAD_POOL
DUMMY_ROW = NPAD_FULL
TBL_FULL = NPAD_FULL + L

_MESH = plsc.VectorSubcoreMesh(core_axis_name="c", subcore_axis_name="s")


def _seg_sum_sc(table, src, dst, npad):
    E = src.shape[0]
    nchunks = E // C
    niters = (nchunks + NW - 1) // NW
    rows_per_sub = npad // NS
    zrows = jnp.zeros((rows_per_sub, D), jnp.float32)

    def body(table_h, src_h, dst_h, zr_h, agg_o,
             agg_sh, src_v, dst_v, rows_v, sem):
        cid = lax.axis_index("c")
        sid = lax.axis_index("s")
        wid = sid * NC + cid
        base = sid * rows_per_sub

        pltpu.sync_copy(zr_h, agg_sh.at[pl.ds(base, rows_per_sub)])
        plsc.subcore_barrier()

        def chunk(j, c):
            cix = wid + j * NW

            @pl.when(cix < nchunks)
            def _go():
                e0 = cix * C
                pltpu.sync_copy(src_h.at[pl.ds(e0, C)], src_v)
                pltpu.sync_copy(dst_h.at[pl.ds(e0, C)], dst_v)
                pltpu.async_copy(table_h.at[src_v], rows_v, sem).wait()
                pltpu.sync_copy(rows_v, agg_sh.at[dst_v], add=True)
            return c
        lax.fori_loop(0, niters, chunk, 0)

        plsc.subcore_barrier()
        row0 = cid * npad + base
        pltpu.sync_copy(agg_sh.at[pl.ds(base, rows_per_sub)],
                        agg_o.at[pl.ds(row0, rows_per_sub)])

    fn = pl.kernel(
        body,
        out_type=jax.ShapeDtypeStruct((NC * npad, D), jnp.float32),
        mesh=_MESH,
        scratch_types=(
            pltpu.VMEM_SHARED((npad, D), jnp.float32),
            pltpu.VMEM((C,), jnp.int32),
            pltpu.VMEM((C,), jnp.int32),
            pltpu.VMEM((C, D), jnp.float32),
            pltpu.SemaphoreType.DMA,
        ),
    )
    return fn(table, src, dst, zrows)


def _deg_sc(dst, npad):
    E = dst.shape[0]
    nchunks = E // C
    niters = (nchunks + NW - 1) // NW
    rows_per_sub = npad // NS
    zrows = jnp.zeros((rows_per_sub, D), jnp.float32)
    ones = jnp.ones((C, D), jnp.float32)

    def body(dst_h, zr_h, on_h, deg_o, deg_sh, dst_v, ones_v, sem):
        cid = lax.axis_index("c")
        sid = lax.axis_index("s")
        wid = sid * NC + cid
        base = sid * rows_per_sub

        pltpu.sync_copy(zr_h, deg_sh.at[pl.ds(base, rows_per_sub)])
        pltpu.sync_copy(on_h, ones_v)
        plsc.subcore_barrier()

        def chunk(j, c):
            cix = wid + j * NW

            @pl.when(cix < nchunks)
            def _go():
                pltpu.sync_copy(dst_h.at[pl.ds(cix * C, C)], dst_v)
                pltpu.sync_copy(ones_v, deg_sh.at[dst_v], add=True)
            return c
        lax.fori_loop(0, niters, chunk, 0)

        plsc.subcore_barrier()
        row0 = cid * npad + base
        pltpu.sync_copy(deg_sh.at[pl.ds(base, rows_per_sub)],
                        deg_o.at[pl.ds(row0, rows_per_sub)])

    fn = pl.kernel(
        body,
        out_type=jax.ShapeDtypeStruct((NC * npad, D), jnp.float32),
        mesh=_MESH,
        scratch_types=(
            pltpu.VMEM_SHARED((npad, D), jnp.float32),
            pltpu.VMEM((C,), jnp.int32),
            pltpu.VMEM((C, D), jnp.float32),
            pltpu.SemaphoreType.DMA,
        ),
    )
    return fn(dst, zrows, ones)


def _unpool_sc(h1, up_pad):
    zrows = jnp.zeros((NPAD_FULL // NS, D), jnp.float32)
    nchunks = UP_PAD // C

    def body(h1_h, up_h, zr_h, out_h, up_v, rows_v, sem):
        cid = lax.axis_index("c")
        sid = lax.axis_index("s")

        @pl.when(cid == 0)
        def _zero():
            pltpu.sync_copy(
                zr_h, out_h.at[pl.ds(sid * (NPAD_FULL // NS), NPAD_FULL // NS)])

        plsc.subcore_barrier()

        def chunk(j, c):
            cix = sid + j * NS

            @pl.when(jnp.logical_and(cid == 0, cix < nchunks))
            def _go():
                e0 = cix * C
                pltpu.sync_copy(up_h.at[pl.ds(e0, C)], up_v)
                pltpu.async_copy(h1_h.at[pl.ds(e0, C)], rows_v, sem).wait()
                pltpu.sync_copy(rows_v, out_h.at[up_v])
            return c
        lax.fori_loop(0, (nchunks + NS - 1) // NS, chunk, 0)

    fn = pl.kernel(
        body,
        out_type=jax.ShapeDtypeStruct((TBL_FULL, D), jnp.float32),
        mesh=_MESH,
        scratch_types=(
            pltpu.VMEM((C,), jnp.int32),
            pltpu.VMEM((C, D), jnp.float32),
            pltpu.SemaphoreType.DMA,
        ),
    )
    return fn(h1, up_pad, zrows)


def _mm_body(x_ref, w_ref, o_ref):
    o_ref[...] = jnp.dot(x_ref[...], w_ref[...],
                         preferred_element_type=jnp.float32)


def _mm(x, w, br):
    n = x.shape[0]
    return pl.pallas_call(
        _mm_body,
        grid=(n // br,),
        in_specs=[pl.BlockSpec((br, D), lambda i: (i, 0)),
                  pl.BlockSpec((D, D), lambda i: (0, 0))],
        out_specs=pl.BlockSpec((br, D), lambda i: (i, 0)),
        out_shape=jax.ShapeDtypeStruct((n, D), jnp.float32),
    )(x, w)


def _finish(aggp, degp, b, w, nvalid, br, out_rows):
    def body(*refs):
        if w is not None:
            agg_ref, deg_ref, b_ref, w_ref, o_ref = refs
        else:
            agg_ref, deg_ref, b_ref, o_ref = refs
        a = agg_ref[0] + agg_ref[1]
        dg = jnp.maximum(deg_ref[0, :, :1] + deg_ref[1, :, :1], 1.0)
        z = jnp.maximum(a / dg + b_ref[...], 0.0)
        if nvalid is not None:
            i = pl.program_id(0)
            row = lax.broadcasted_iota(jnp.int32, (br, D), 0) + i * br
            z = jnp.where(row < nvalid, z, 0.0)
        if w is not None:
            o_ref[...] = jnp.dot(z, w_ref[...],
                                 preferred_element_type=jnp.float32)
        else:
            o_ref[...] = z

    in_specs = [pl.BlockSpec((NC, br, D), lambda i: (0, i, 0)),
                pl.BlockSpec((NC, br, D), lambda i: (0, i, 0)),
                pl.BlockSpec((1, D), lambda i: (0, 0))]
    args = [aggp, degp, b]
    if w is not None:
        in_specs.append(pl.BlockSpec((D, D), lambda i: (0, 0)))
        args.append(w)
    return pl.pallas_call(
        body,
        grid=(out_rows // br,),
        in_specs=in_specs,
        out_specs=pl.BlockSpec((br, D), lambda i: (i, 0)),
        out_shape=jax.ShapeDtypeStruct((out_rows, D), jnp.float32),
    )(*args)


def kernel(x, edge_index, pooled_x, pooled_edge_index, unpool_info,
           W0, b0, W1, b1, W2, b2):
    src_p, dst_p = pooled_edge_index[0], pooled_edge_index[1]
    src_f, dst_f = edge_index[0], edge_index[1]
    up_pad = jnp.concatenate(
        [unpool_info,
         jnp.full((UP_PAD - N_POOL,), DUMMY_ROW, jnp.int32)])

    h0 = _mm(pooled_x, W0, 1000)
    agg0 = _seg_sum_sc(h0, src_p, dst_p, NPAD_POOL).reshape(NC, NPAD_POOL, D)
    deg0 = _deg_sc(dst_p, NPAD_POOL).reshape(NC, NPAD_POOL, D)
    h1 = _finish(agg0, deg0, b0.reshape(1, D), W1, N_POOL, 640, NPAD_POOL)

    h1f = _unpool_sc(h1, up_pad)

    agg1 = _seg_sum_sc(h1f, src_f, dst_f, NPAD_FULL).reshape(NC, NPAD_FULL, D)
    deg1 = _deg_sc(dst_f, NPAD_FULL).reshape(NC, NPAD_FULL, D)
    h2 = _finish(agg1, deg1, b1.reshape(1, D), W2, None, 1280, NPAD_FULL)

    agg2 = _seg_sum_sc(h2, src_f, dst_f, NPAD_FULL).reshape(NC, NPAD_FULL, D)
    return _finish(agg2, deg1, b2.reshape(1, D), None, None, 1000, N_FULL)
